# Initial kernel scaffold; baseline (speedup 1.0000x reference)
import functools

import jax
import jax.numpy as jnp
from jax import lax
from jax.experimental import pallas as pl
from jax.experimental.pallas import tpu as pltpu
from jax.experimental.pallas import tpu_sc as plsc

N = 10000
E = 320000
D = 128
G = 64
NE = 7

NC = 2
NS = 16
NW = NC * NS

C = 80
ROWS = E // C
RPW = ROWS // NW
RPT = N // NS

_mesh = plsc.VectorSubcoreMesh(core_axis_name="c", subcore_axis_name="s")


@functools.partial(
    pl.kernel,
    out_type=jax.ShapeDtypeStruct((NC, N, D), jnp.float32),
    mesh=_mesh,
    scratch_types=[
        pltpu.VMEM((RPW, C), jnp.int32),
        pltpu.VMEM((RPW, C), jnp.int32),
        pltpu.VMEM((C, D), jnp.float32),
        pltpu.VMEM_SHARED((N, D), jnp.float32),
        pltpu.SemaphoreType.DMA,
    ],
)
def _sc_segsum(h_hbm, src_hbm, dst_hbm, zeros_hbm, out_hbm,
               src_v, dst_v, rows_v, agg_sh, sem):
    cid = lax.axis_index("c")
    sid = lax.axis_index("s")
    wid = sid * NC + cid

    pltpu.sync_copy(zeros_hbm.at[pl.ds(sid * RPT, RPT)],
                    agg_sh.at[pl.ds(sid * RPT, RPT)])
    pltpu.sync_copy(src_hbm.at[pl.ds(wid * RPW, RPW)], src_v)
    pltpu.sync_copy(dst_hbm.at[pl.ds(wid * RPW, RPW)], dst_v)
    plsc.subcore_barrier()

    def body(j, carry):
        pltpu.async_copy(h_hbm.at[src_v.at[j]], rows_v, sem).wait()
        pltpu.sync_copy(rows_v, agg_sh.at[dst_v.at[j]], add=True)
        return carry

    lax.fori_loop(0, RPW, body, 0)
    plsc.subcore_barrier()

    pltpu.sync_copy(agg_sh.at[pl.ds(sid * RPT, RPT)],
                    out_hbm.at[cid, pl.ds(sid * RPT, RPT)])


@functools.partial(
    pl.kernel,
    out_type=jax.ShapeDtypeStruct((NC, N, 16), jnp.float32),
    mesh=_mesh,
    scratch_types=[
        pltpu.VMEM((RPW, C), jnp.int32),
        pltpu.VMEM((C, 16), jnp.float32),
        pltpu.VMEM_SHARED((N, 16), jnp.float32),
    ],
)
def _sc_counts(dst_hbm, zeros_hbm, out_hbm, dst_v, ones_v, cnt_sh):
    cid = lax.axis_index("c")
    sid = lax.axis_index("s")
    wid = sid * NC + cid

    pltpu.sync_copy(zeros_hbm.at[pl.ds(sid * RPT, RPT)],
                    cnt_sh.at[pl.ds(sid * RPT, RPT)])
    pltpu.sync_copy(dst_hbm.at[pl.ds(wid * RPW, RPW)], dst_v)

    def init_ones(r, carry):
        ones_v[r] = jnp.full((16,), 1.0, dtype=jnp.float32)
        return carry

    lax.fori_loop(0, C, init_ones, 0)
    plsc.subcore_barrier()

    def body(j, carry):
        pltpu.sync_copy(ones_v, cnt_sh.at[dst_v.at[j]], add=True)
        return carry

    lax.fori_loop(0, RPW, body, 0)
    plsc.subcore_barrier()

    pltpu.sync_copy(cnt_sh.at[pl.ds(sid * RPT, RPT)],
                    out_hbm.at[cid, pl.ds(sid * RPT, RPT)])


def _dense_body(agg_ref, cnt_ref, h_ref, wlT_ref, bl_ref, wrT_ref,
                gamma_ref, beta_ref, out_ref):
    agg = agg_ref[0] + agg_ref[1]
    cnt = (cnt_ref[0] + cnt_ref[1])[:, 0:1]
    mean = agg / jnp.maximum(cnt, 1.0)
    h = h_ref[...]
    y = (jnp.dot(mean, wlT_ref[...], preferred_element_type=jnp.float32)
         + bl_ref[...]
         + jnp.dot(h, wrT_ref[...], preferred_element_type=jnp.float32))
    mu = jnp.mean(y, axis=0, keepdims=True)
    var = jnp.mean((y - mu) * (y - mu), axis=0, keepdims=True)
    z = (y - mu) * lax.rsqrt(var + 1e-5) * gamma_ref[...] + beta_ref[...]
    out_ref[...] = jnp.maximum(z, 0.0)


_tc_dense = pl.pallas_call(
    _dense_body,
    out_shape=jax.ShapeDtypeStruct((N, D), jnp.float32),
)


def _final_body(h_ref, batch_ref, wfcT_ref, bfc_ref, scores_ref,
                probs_ref, ill_ref):
    h = h_ref[...]
    gids = lax.broadcasted_iota(jnp.int32, (N, G), 1)
    onehot = (batch_ref[...] == gids).astype(jnp.float32)
    pooled_sum = lax.dot_general(onehot, h, (((0,), (0,)), ((), ())),
                                 preferred_element_type=jnp.float32)
    counts = jnp.sum(onehot, axis=0)[:, None]
    pooled = pooled_sum / jnp.maximum(counts, 1.0)
    logits = jnp.dot(pooled, wfcT_ref[...],
                     preferred_element_type=jnp.float32) + bfc_ref[...]
    probs = jax.nn.sigmoid(logits)
    probs_ref[...] = probs
    ill = jnp.sum(probs * scores_ref[...], axis=1)[:, None]
    ill_ref[...] = jnp.broadcast_to(ill, (G, D))


_tc_final = pl.pallas_call(
    _final_body,
    out_shape=(jax.ShapeDtypeStruct((G, D), jnp.float32),
               jax.ShapeDtypeStruct((G, D), jnp.float32)),
)


def kernel(x, edge_index, batch,
           W_l0, b_l0, W_r0, gamma0, beta0,
           W_l1, b_l1, W_r1, gamma1, beta1,
           W_l2, b_l2, W_r2, gamma2, beta2,
           W_fc, b_fc):
    src = edge_index[0].reshape(ROWS, C)
    dst = edge_index[1].reshape(ROWS, C)
    zeros128 = jnp.zeros((N, D), jnp.float32)
    zeros16 = jnp.zeros((N, 16), jnp.float32)

    cnt_p = _sc_counts(dst, zeros16)

    params = [
        (W_l0, b_l0, W_r0, gamma0, beta0),
        (W_l1, b_l1, W_r1, gamma1, beta1),
        (W_l2, b_l2, W_r2, gamma2, beta2),
    ]
    h = x
    for (W_l, b_l, W_r, gamma, beta) in params:
        agg_p = _sc_segsum(h, src, dst, zeros128)
        h = _tc_dense(agg_p, cnt_p, h, W_l.T, b_l[None, :], W_r.T,
                      gamma[None, :], beta[None, :])

    wfcT_pad = jnp.zeros((D, D), jnp.float32).at[:, :NE].set(W_fc.T)
    bfc_pad = jnp.zeros((1, D), jnp.float32).at[0, :NE].set(b_fc)
    scores = jnp.array([1.0, 0.5, 0.8, -1.0, 1.0, -0.8, 0.0],
                       dtype=jnp.float32)
    scores_pad = jnp.zeros((1, D), jnp.float32).at[0, :NE].set(scores)

    probs_pad, ill_pad = _tc_final(h, batch[:, None], wfcT_pad, bfc_pad,
                                   scores_pad)
    return probs_pad[:, :NE], ill_pad[:, 0]


# speedup vs baseline: 6.3476x; 6.3476x over previous
import functools

import jax
import jax.numpy as jnp
from jax import lax
from jax.experimental import pallas as pl
from jax.experimental.pallas import tpu as pltpu
from jax.experimental.pallas import tpu_sc as plsc

N = 10000
E = 320000
D = 128
G = 64
NE = 7

NC = 2
NS = 16
NW = NC * NS

C = 80
ROWS = E // C
RPW = ROWS // NW
RPT = 624
REM = N - NS * RPT

@functools.lru_cache(maxsize=None)
def _sc_kernels():
    _mesh = plsc.VectorSubcoreMesh(core_axis_name="c", subcore_axis_name="s",
                                   num_cores=NC, num_subcores=NS)


    @functools.partial(
        pl.kernel,
        out_type=jax.ShapeDtypeStruct((NC, N, D), jnp.float32),
        mesh=_mesh,
        scratch_types=[
            pltpu.VMEM((RPW, C), jnp.int32),
            pltpu.VMEM((RPW, C), jnp.int32),
            pltpu.VMEM((C, D), jnp.float32),
            pltpu.VMEM_SHARED((N, D), jnp.float32),
        ],
    )
    def _sc_segsum(h_hbm, src_hbm, dst_hbm, zeros_hbm, out_hbm,
                   src_v, dst_v, rows_v, agg_sh):
        cid = lax.axis_index("c")
        sid = lax.axis_index("s")
        wid = sid * NC + cid

        pltpu.sync_copy(zeros_hbm.at[pl.ds(sid * RPT, RPT)],
                        agg_sh.at[pl.ds(sid * RPT, RPT)])

        @pl.when(sid == 0)
        def _():
            pltpu.sync_copy(zeros_hbm.at[pl.ds(NS * RPT, REM)],
                            agg_sh.at[pl.ds(NS * RPT, REM)])

        pltpu.sync_copy(src_hbm.at[wid], src_v)
        pltpu.sync_copy(dst_hbm.at[wid], dst_v)
        plsc.subcore_barrier()

        def body(j, carry):
            pltpu.sync_copy(h_hbm.at[src_v.at[j]], rows_v)
            pltpu.sync_copy(rows_v, agg_sh.at[dst_v.at[j]], add=True)
            return carry

        lax.fori_loop(0, RPW, body, 0)
        plsc.subcore_barrier()

        pltpu.sync_copy(agg_sh.at[pl.ds(sid * RPT, RPT)],
                        out_hbm.at[cid, pl.ds(sid * RPT, RPT)])

        @pl.when(sid == 0)
        def _():
            pltpu.sync_copy(agg_sh.at[pl.ds(NS * RPT, REM)],
                            out_hbm.at[cid, pl.ds(NS * RPT, REM)])


    @functools.partial(
        pl.kernel,
        out_type=jax.ShapeDtypeStruct((NC, N, D), jnp.float32),
        mesh=_mesh,
        scratch_types=[
            pltpu.VMEM((RPW, C), jnp.int32),
            pltpu.VMEM((C, D), jnp.float32),
            pltpu.VMEM_SHARED((N, D), jnp.float32),
        ],
    )
    def _sc_counts(dst_hbm, zeros_hbm, ones_hbm, out_hbm, dst_v, ones_v,
                   cnt_sh):
        cid = lax.axis_index("c")
        sid = lax.axis_index("s")
        wid = sid * NC + cid

        pltpu.sync_copy(zeros_hbm.at[pl.ds(sid * RPT, RPT)],
                        cnt_sh.at[pl.ds(sid * RPT, RPT)])

        @pl.when(sid == 0)
        def _():
            pltpu.sync_copy(zeros_hbm.at[pl.ds(NS * RPT, REM)],
                            cnt_sh.at[pl.ds(NS * RPT, REM)])

        pltpu.sync_copy(dst_hbm.at[wid], dst_v)
        pltpu.sync_copy(ones_hbm, ones_v)
        plsc.subcore_barrier()

        def body(j, carry):
            pltpu.sync_copy(ones_v, cnt_sh.at[dst_v.at[j]], add=True)
            return carry

        lax.fori_loop(0, RPW, body, 0)
        plsc.subcore_barrier()

        pltpu.sync_copy(cnt_sh.at[pl.ds(sid * RPT, RPT)],
                        out_hbm.at[cid, pl.ds(sid * RPT, RPT)])

        @pl.when(sid == 0)
        def _():
            pltpu.sync_copy(cnt_sh.at[pl.ds(NS * RPT, REM)],
                            out_hbm.at[cid, pl.ds(NS * RPT, REM)])

    return _sc_segsum, _sc_counts


def _dense_body(agg_ref, cnt_ref, h_ref, wlT_ref, bl_ref, wrT_ref,
                gamma_ref, beta_ref, out_ref):
    agg = agg_ref[0] + agg_ref[1]
    cnt = (cnt_ref[0] + cnt_ref[1])[:, 0:1]
    mean = agg / jnp.maximum(cnt, 1.0)
    h = h_ref[...]
    y = (jnp.dot(mean, wlT_ref[...], preferred_element_type=jnp.float32)
         + bl_ref[...]
         + jnp.dot(h, wrT_ref[...], preferred_element_type=jnp.float32))
    mu = jnp.mean(y, axis=0, keepdims=True)
    var = jnp.mean((y - mu) * (y - mu), axis=0, keepdims=True)
    z = (y - mu) * lax.rsqrt(var + 1e-5) * gamma_ref[...] + beta_ref[...]
    out_ref[...] = jnp.maximum(z, 0.0)


_tc_dense = pl.pallas_call(
    _dense_body,
    out_shape=jax.ShapeDtypeStruct((N, D), jnp.float32),
)


def _final_body(h_ref, batch_ref, wfcT_ref, bfc_ref, scores_ref,
                probs_ref, ill_ref):
    h = h_ref[...]
    gids = lax.broadcasted_iota(jnp.int32, (N, G), 1)
    onehot = (batch_ref[...] == gids).astype(jnp.float32)
    pooled_sum = lax.dot_general(onehot, h, (((0,), (0,)), ((), ())),
                                 preferred_element_type=jnp.float32)
    counts = jnp.sum(onehot, axis=0)[:, None]
    pooled = pooled_sum / jnp.maximum(counts, 1.0)
    logits = jnp.dot(pooled, wfcT_ref[...],
                     preferred_element_type=jnp.float32) + bfc_ref[...]
    probs = jax.nn.sigmoid(logits)
    probs_ref[...] = probs
    ill = jnp.sum(probs * scores_ref[...], axis=1)[:, None]
    ill_ref[...] = jnp.broadcast_to(ill, (G, D))


_tc_final = pl.pallas_call(
    _final_body,
    out_shape=(jax.ShapeDtypeStruct((G, D), jnp.float32),
               jax.ShapeDtypeStruct((G, D), jnp.float32)),
)


def kernel(x, edge_index, batch,
           W_l0, b_l0, W_r0, gamma0, beta0,
           W_l1, b_l1, W_r1, gamma1, beta1,
           W_l2, b_l2, W_r2, gamma2, beta2,
           W_fc, b_fc):
    src = edge_index[0].reshape(NW, RPW, C)
    dst = edge_index[1].reshape(NW, RPW, C)
    zeros128 = jnp.zeros((N, D), jnp.float32)


    _sc_segsum, _sc_counts = _sc_kernels()
    ones128 = jnp.ones((C, D), jnp.float32)
    cnt_p = _sc_counts(dst, zeros128, ones128)

    params = [
        (W_l0, b_l0, W_r0, gamma0, beta0),
        (W_l1, b_l1, W_r1, gamma1, beta1),
        (W_l2, b_l2, W_r2, gamma2, beta2),
    ]
    h = x
    for (W_l, b_l, W_r, gamma, beta) in params:
        agg_p = _sc_segsum(h, src, dst, zeros128)
        h = _tc_dense(agg_p, cnt_p, h, W_l.T, b_l[None, :], W_r.T,
                      gamma[None, :], beta[None, :])

    wfcT_pad = jnp.zeros((D, D), jnp.float32).at[:, :NE].set(W_fc.T)
    bfc_pad = jnp.zeros((1, D), jnp.float32).at[0, :NE].set(b_fc)
    scores = jnp.array([1.0, 0.5, 0.8, -1.0, 1.0, -0.8, 0.0],
                       dtype=jnp.float32)
    scores_pad = jnp.zeros((1, D), jnp.float32).at[0, :NE].set(scores)

    probs_pad, ill_pad = _tc_final(h, batch[:, None], wfcT_pad, bfc_pad,
                                   scores_pad)
    return probs_pad[:, :NE], ill_pad[:, 0]
